# Initial kernel scaffold; baseline (speedup 1.0000x reference)
import jax
import jax.numpy as jnp
from jax.experimental import pallas as pl


def kernel(adv_pcs):
    raise NotImplementedError("write your pallas kernel here")


# speedup vs baseline: 38.0731x; 38.0731x over previous
import functools

import numpy as np
import jax
import jax.numpy as jnp
from jax import lax
from jax.experimental import pallas as pl
from jax.experimental.pallas import tpu as pltpu
from jax.experimental.pallas import tpu_sc as plsc

_B = 16
_N = 2048
_NPOINT = 102
_HALF = _NPOINT // 2
_NCHUNKS = _N // 16
_L = 16

_PERCENTAGES = [0.004, 0.006, 0.008, 0.01, 0.012]
_LEVELS = []
for _p in _PERCENTAGES:
    _p4 = _p * 4
    _ns = int(_N * _p4)
    _pad = ((_ns + _L - 1) // _L) * _L
    _r = float(np.sqrt(_p4 * 1.0))
    _thr = np.float32(_r * _r)
    _expect = np.float32(np.sqrt(np.float32(np.pi * 1.0 / _ns * _p4)))
    _w = np.float32((_p4 * 100.0) ** 2
                    / (_NPOINT * _ns)
                    / (float(_expect) + 1e-12)
                    / len(_PERCENTAGES))
    _LEVELS.append((_ns, _pad, _thr, _expect, _w))
_THR_MAX = _LEVELS[-1][2]
_BIG = np.float32(1e30)
_SENT = np.float32(1e3)


def _bf16_round(x):
    i = plsc.bitcast(x, jnp.int32)
    r = (i + jnp.int32(0x7FFF) + ((i >> 16) & jnp.int32(1))) & jnp.int32(-65536)
    return plsc.bitcast(r, jnp.float32)


def _vsqrt(x):
    i = plsc.bitcast(x, jnp.int32)
    y = plsc.bitcast(jnp.int32(0x5F3759DF) - (i >> 1), jnp.float32)
    for _ in range(3):
        y = y * (jnp.float32(1.5) - jnp.float32(0.5) * x * y * y)
    return x * y


def _body(x_hbm, y_hbm, z_hbm, out_hbm,
          xv, yv, zv, dminv, fidxv, cxv, cyv, czv,
          gdv, giv, hiv, hxv, hyv, hzv, hsv, accv):
    wid = lax.axis_index("s") * 2 + lax.axis_index("c")
    b = wid >> 1
    c0 = (wid & 1) * _HALF
    lane = lax.iota(jnp.int32, _L)

    pltpu.sync_copy(x_hbm.at[b], xv)
    pltpu.sync_copy(y_hbm.at[b], yv)
    pltpu.sync_copy(z_hbm.at[b], zv)

    def _init(i, _):
        sl = pl.ds(i * _L, _L)
        dminv[sl] = jnp.full((_L,), 1e10, jnp.float32)
        return 0
    lax.fori_loop(0, _NCHUNKS, _init, 0)
    for k in range(128 // _L):
        z16 = jnp.zeros((_L,), jnp.int32)
        fidxv[pl.ds(k * _L, _L)] = z16
        hiv[pl.ds(k * _L, _L)] = z16

    def _fps_step(s, far):
        far_splat = jnp.full((_L,), 0, jnp.int32) + far
        cxs = plsc.load_gather(xv, [far_splat])
        cys = plsc.load_gather(yv, [far_splat])
        czs = plsc.load_gather(zv, [far_splat])
        plsc.store_scatter(fidxv, [jnp.full((_L,), 0, jnp.int32) + s],
                           far_splat, mask=lane == 0)

        def _chunk(i, carry):
            bv, bi = carry
            sl = pl.ds(i * _L, _L)
            dx = xv[sl] - cxs
            dy = yv[sl] - cys
            dz = zv[sl] - czs
            d = dx * dx + dy * dy + dz * dz
            dm = jnp.minimum(dminv[sl], d)
            dminv[sl] = dm
            upd = dm > bv
            bv = jnp.where(upd, dm, bv)
            bi = jnp.where(upd, lane + i * _L, bi)
            return bv, bi

        bv0 = jnp.full((_L,), -1.0, jnp.float32)
        bi0 = jnp.zeros((_L,), jnp.int32)
        bv, bi = lax.fori_loop(0, _NCHUNKS, _chunk, (bv0, bi0))
        m = jnp.max(bv)
        cand = jnp.where(bv == m, bi, jnp.int32(2**31 - 1))
        return jnp.min(cand)

    lax.fori_loop(0, _NPOINT, _fps_step, jnp.int32(0))

    for k in range(128 // _L):
        sl = pl.ds(k * _L, _L)
        idxs = fidxv[sl]
        cxv[sl] = plsc.load_gather(xv, [idxs])
        cyv[sl] = plsc.load_gather(yv, [idxs])
        czv[sl] = plsc.load_gather(zv, [idxs])

    def _center(ci, acc):
        c_splat = jnp.full((_L,), 0, jnp.int32) + (c0 + ci)
        ccx = plsc.load_gather(cxv, [c_splat])
        ccy = plsc.load_gather(cyv, [c_splat])
        ccz = plsc.load_gather(czv, [c_splat])

        def _pa(i, off):
            sl = pl.ds(i * _L, _L)
            dx = xv[sl] - ccx
            dy = yv[sl] - ccy
            dz = zv[sl] - ccz
            d = dx * dx + dy * dy + dz * dz
            msk = d <= _THR_MAX
            mi = msk.astype(jnp.int32)
            pos = off + jnp.cumsum(mi) - 1
            plsc.store_scatter(gdv, [pos], d, mask=msk)
            plsc.store_scatter(giv, [pos], lane + i * _L, mask=msk)
            return off + jnp.sum(mi)

        cnt = lax.fori_loop(0, _NCHUNKS, _pa, jnp.int32(0))
        nch = (cnt + (_L - 1)) // _L

        for ns, pad, thr, expect, w in _LEVELS:
            def _pb(i, off2, _thr=thr, _ns=ns):
                sl = pl.ds(i * _L, _L)
                d = gdv[sl]
                gi = giv[sl]
                msk = ((lane + i * _L) < cnt) & (d <= _thr) & (off2 < _ns)
                mi = msk.astype(jnp.int32)
                pos = off2 + jnp.cumsum(mi) - 1
                plsc.store_scatter(hiv, [pos], gi, mask=msk)
                return off2 + jnp.sum(mi)

            cnt2 = lax.fori_loop(0, nch, _pb, jnp.int32(0))
            cntp = jnp.minimum(cnt2, ns)

            for k in range(pad // _L):
                sl = pl.ds(k * _L, _L)
                lidx = lane + k * _L
                vi = plsc.load_gather(
                    hiv, [jnp.where(lidx < cntp, lidx, jnp.int32(0))])
                px = plsc.load_gather(xv, [vi])
                py = plsc.load_gather(yv, [vi])
                pz = plsc.load_gather(zv, [vi])
                inb = lidx < ns
                px = jnp.where(inb, px, _SENT)
                py = jnp.where(inb, py, _SENT)
                pz = jnp.where(inb, pz, _SENT)
                hsv[sl] = px * px + py * py + pz * pz
                hxv[sl] = _bf16_round(px)
                hyv[sl] = _bf16_round(py)
                hzv[sl] = _bf16_round(pz)

            for k in range(pad // _L):
                sl = pl.ds(k * _L, _L)
                gxi = hxv[sl]
                gyi = hyv[sl]
                gzi = hzv[sl]
                sqi = hsv[sl]
                lidx = lane + k * _L

                def _nnj(j, carry, _gxi=gxi, _gyi=gyi, _gzi=gzi, _sqi=sqi):
                    m1, m2 = carry
                    j_splat = jnp.full((_L,), 0, jnp.int32) + j
                    bjx = plsc.load_gather(hxv, [j_splat])
                    bjy = plsc.load_gather(hyv, [j_splat])
                    bjz = plsc.load_gather(hzv, [j_splat])
                    sqj = plsc.load_gather(hsv, [j_splat])
                    dot = _gxi * bjx + _gyi * bjy + _gzi * bjz
                    d = (_sqi + sqj) - (dot + dot)
                    d = jnp.maximum(d, jnp.float32(0.0))
                    m2 = jnp.minimum(m2, jnp.maximum(m1, d))
                    m1 = jnp.minimum(m1, d)
                    return m1, m2

                m1, m2 = lax.fori_loop(
                    0, ns, _nnj,
                    (jnp.full((_L,), _BIG, jnp.float32),
                     jnp.full((_L,), _BIG, jnp.float32)))
                nn = _vsqrt(jnp.maximum(m2, jnp.float32(1e-12)))
                cb = nn - expect
                cb = cb * cb
                cb = jnp.where(lidx < ns, cb, jnp.float32(0.0))
                acc = acc + w * cb
        return acc

    acc = lax.fori_loop(0, _HALF, _center,
                        jnp.zeros((_L,), jnp.float32))
    accv[...] = acc
    pltpu.sync_copy(accv, out_hbm.at[wid])


@jax.jit
def _loss_sc(x, y, z):
    mesh = plsc.VectorSubcoreMesh(core_axis_name="c", subcore_axis_name="s")
    f = pl.kernel(
        _body,
        out_type=jax.ShapeDtypeStruct((32, _L), jnp.float32),
        mesh=mesh,
        compiler_params=pltpu.CompilerParams(needs_layout_passes=False),
        scratch_types=[
            pltpu.VMEM((_N,), jnp.float32),
            pltpu.VMEM((_N,), jnp.float32),
            pltpu.VMEM((_N,), jnp.float32),
            pltpu.VMEM((_N,), jnp.float32),
            pltpu.VMEM((128,), jnp.int32),
            pltpu.VMEM((128,), jnp.float32),
            pltpu.VMEM((128,), jnp.float32),
            pltpu.VMEM((128,), jnp.float32),
            pltpu.VMEM((_N,), jnp.float32),
            pltpu.VMEM((_N,), jnp.int32),
            pltpu.VMEM((128,), jnp.int32),
            pltpu.VMEM((128,), jnp.float32),
            pltpu.VMEM((128,), jnp.float32),
            pltpu.VMEM((128,), jnp.float32),
            pltpu.VMEM((128,), jnp.float32),
            pltpu.VMEM((_L,), jnp.float32),
        ],
    )
    return f(x, y, z)


def kernel(adv_pcs):
    x = adv_pcs[:, :, 0]
    y = adv_pcs[:, :, 1]
    z = adv_pcs[:, :, 2]
    parts = _loss_sc(x, y, z)
    return parts.reshape(_B, 2, _L).sum(axis=(1, 2))


# speedup vs baseline: 41.3494x; 1.0861x over previous
import functools

import numpy as np
import jax
import jax.numpy as jnp
from jax import lax
from jax.experimental import pallas as pl
from jax.experimental.pallas import tpu as pltpu
from jax.experimental.pallas import tpu_sc as plsc

_B = 16
_N = 2048
_NPOINT = 102
_HALF = _NPOINT // 2
_NCHUNKS = _N // 16
_L = 16

_PERCENTAGES = [0.004, 0.006, 0.008, 0.01, 0.012]
_LEVELS = []
for _p in _PERCENTAGES:
    _p4 = _p * 4
    _ns = int(_N * _p4)
    _pad = ((_ns + _L - 1) // _L) * _L
    _r = float(np.sqrt(_p4 * 1.0))
    _thr = np.float32(_r * _r)
    _expect = np.float32(np.sqrt(np.float32(np.pi * 1.0 / _ns * _p4)))
    _w = np.float32((_p4 * 100.0) ** 2
                    / (_NPOINT * _ns)
                    / (float(_expect) + 1e-12)
                    / len(_PERCENTAGES))
    _LEVELS.append((_ns, _pad, _thr, _expect, _w))
_THR_MAX = _LEVELS[-1][2]
_BIG = np.float32(1e30)
_SENT = np.float32(1e3)


def _bf16_round(x):
    i = plsc.bitcast(x, jnp.int32)
    r = (i + jnp.int32(0x7FFF) + ((i >> 16) & jnp.int32(1))) & jnp.int32(-65536)
    return plsc.bitcast(r, jnp.float32)


def _vsqrt(x):
    i = plsc.bitcast(x, jnp.int32)
    y = plsc.bitcast(jnp.int32(0x5F3759DF) - (i >> 1), jnp.float32)
    for _ in range(3):
        y = y * (jnp.float32(1.5) - jnp.float32(0.5) * x * y * y)
    return x * y


def _body(x_hbm, y_hbm, z_hbm, out_hbm,
          xv, yv, zv, dminv, fidxv, cxv, cyv, czv,
          gdv, giv, hiv, hxv, hyv, hzv, hsv, accv):
    wid = lax.axis_index("s") * 2 + lax.axis_index("c")
    b = wid >> 1
    c0 = (wid & 1) * _HALF
    lane = lax.iota(jnp.int32, _L)

    pltpu.sync_copy(x_hbm.at[b], xv)
    pltpu.sync_copy(y_hbm.at[b], yv)
    pltpu.sync_copy(z_hbm.at[b], zv)

    def _init(i, _):
        sl = pl.ds(i * _L, _L)
        dminv[sl] = jnp.full((_L,), 1e10, jnp.float32)
        return 0
    lax.fori_loop(0, _NCHUNKS, _init, 0)
    for k in range(128 // _L):
        z16 = jnp.zeros((_L,), jnp.int32)
        fidxv[pl.ds(k * _L, _L)] = z16
        hiv[pl.ds(k * _L, _L)] = z16

    def _fps_step(s, far):
        far_splat = jnp.full((_L,), 0, jnp.int32) + far
        cxs = plsc.load_gather(xv, [far_splat])
        cys = plsc.load_gather(yv, [far_splat])
        czs = plsc.load_gather(zv, [far_splat])
        plsc.store_scatter(fidxv, [jnp.full((_L,), 0, jnp.int32) + s],
                           far_splat, mask=lane == 0)

        def _chunk(i, carry):
            bv, bi = carry
            sl = pl.ds(i * _L, _L)
            dx = xv[sl] - cxs
            dy = yv[sl] - cys
            dz = zv[sl] - czs
            d = dx * dx + dy * dy + dz * dz
            dm = jnp.minimum(dminv[sl], d)
            dminv[sl] = dm
            upd = dm > bv
            bv = jnp.where(upd, dm, bv)
            bi = jnp.where(upd, lane + i * _L, bi)
            return bv, bi

        bv0 = jnp.full((_L,), -1.0, jnp.float32)
        bi0 = jnp.zeros((_L,), jnp.int32)
        bv, bi = lax.fori_loop(0, _NCHUNKS, _chunk, (bv0, bi0), unroll=4)
        m = jnp.max(bv)
        cand = jnp.where(bv == m, bi, jnp.int32(2**31 - 1))
        return jnp.min(cand)

    lax.fori_loop(0, _NPOINT, _fps_step, jnp.int32(0))

    for k in range(128 // _L):
        sl = pl.ds(k * _L, _L)
        idxs = fidxv[sl]
        cxv[sl] = plsc.load_gather(xv, [idxs])
        cyv[sl] = plsc.load_gather(yv, [idxs])
        czv[sl] = plsc.load_gather(zv, [idxs])

    def _center(ci, acc):
        c_splat = jnp.full((_L,), 0, jnp.int32) + (c0 + ci)
        ccx = plsc.load_gather(cxv, [c_splat])
        ccy = plsc.load_gather(cyv, [c_splat])
        ccz = plsc.load_gather(czv, [c_splat])

        def _pa(i, off):
            sl = pl.ds(i * _L, _L)
            dx = xv[sl] - ccx
            dy = yv[sl] - ccy
            dz = zv[sl] - ccz
            d = dx * dx + dy * dy + dz * dz
            msk = d <= _THR_MAX
            mi = msk.astype(jnp.int32)
            pos = off + jnp.cumsum(mi) - 1
            plsc.store_scatter(gdv, [pos], d, mask=msk)
            plsc.store_scatter(giv, [pos], lane + i * _L, mask=msk)
            return off + jnp.sum(mi)

        cnt = lax.fori_loop(0, _NCHUNKS, _pa, jnp.int32(0), unroll=4)
        nch = (cnt + (_L - 1)) // _L

        for ns, pad, thr, expect, w in _LEVELS:
            def _pb(i, off2, _thr=thr, _ns=ns):
                sl = pl.ds(i * _L, _L)
                d = gdv[sl]
                gi = giv[sl]
                msk = ((lane + i * _L) < cnt) & (d <= _thr) & (off2 < _ns)
                mi = msk.astype(jnp.int32)
                pos = off2 + jnp.cumsum(mi) - 1
                plsc.store_scatter(hiv, [pos], gi, mask=msk)
                return off2 + jnp.sum(mi)

            cnt2 = lax.fori_loop(0, nch, _pb, jnp.int32(0))
            cntp = jnp.minimum(cnt2, ns)

            for k in range(pad // _L):
                sl = pl.ds(k * _L, _L)
                lidx = lane + k * _L
                vi = plsc.load_gather(
                    hiv, [jnp.where(lidx < cntp, lidx, jnp.int32(0))])
                px = plsc.load_gather(xv, [vi])
                py = plsc.load_gather(yv, [vi])
                pz = plsc.load_gather(zv, [vi])
                inb = lidx < ns
                px = jnp.where(inb, px, _SENT)
                py = jnp.where(inb, py, _SENT)
                pz = jnp.where(inb, pz, _SENT)
                hsv[sl] = px * px + py * py + pz * pz
                hxv[sl] = _bf16_round(px)
                hyv[sl] = _bf16_round(py)
                hzv[sl] = _bf16_round(pz)

            for k in range(pad // _L):
                sl = pl.ds(k * _L, _L)
                gxi = hxv[sl]
                gyi = hyv[sl]
                gzi = hzv[sl]
                sqi = hsv[sl]
                lidx = lane + k * _L

                def _nnj(j, carry, _gxi=gxi, _gyi=gyi, _gzi=gzi, _sqi=sqi):
                    m1, m2 = carry
                    j_splat = jnp.full((_L,), 0, jnp.int32) + j
                    bjx = plsc.load_gather(hxv, [j_splat])
                    bjy = plsc.load_gather(hyv, [j_splat])
                    bjz = plsc.load_gather(hzv, [j_splat])
                    sqj = plsc.load_gather(hsv, [j_splat])
                    dot = _gxi * bjx + _gyi * bjy + _gzi * bjz
                    d = (_sqi + sqj) - (dot + dot)
                    d = jnp.maximum(d, jnp.float32(0.0))
                    m2 = jnp.minimum(m2, jnp.maximum(m1, d))
                    m1 = jnp.minimum(m1, d)
                    return m1, m2

                m1, m2 = lax.fori_loop(
                    0, ns, _nnj,
                    (jnp.full((_L,), _BIG, jnp.float32),
                     jnp.full((_L,), _BIG, jnp.float32)),
                    unroll=7)
                nn = _vsqrt(jnp.maximum(m2, jnp.float32(1e-12)))
                cb = nn - expect
                cb = cb * cb
                cb = jnp.where(lidx < ns, cb, jnp.float32(0.0))
                acc = acc + w * cb
        return acc

    acc = lax.fori_loop(0, _HALF, _center,
                        jnp.zeros((_L,), jnp.float32))
    accv[...] = acc
    pltpu.sync_copy(accv, out_hbm.at[wid])


@jax.jit
def _loss_sc(x, y, z):
    mesh = plsc.VectorSubcoreMesh(core_axis_name="c", subcore_axis_name="s")
    f = pl.kernel(
        _body,
        out_type=jax.ShapeDtypeStruct((32, _L), jnp.float32),
        mesh=mesh,
        compiler_params=pltpu.CompilerParams(needs_layout_passes=False),
        scratch_types=[
            pltpu.VMEM((_N,), jnp.float32),
            pltpu.VMEM((_N,), jnp.float32),
            pltpu.VMEM((_N,), jnp.float32),
            pltpu.VMEM((_N,), jnp.float32),
            pltpu.VMEM((128,), jnp.int32),
            pltpu.VMEM((128,), jnp.float32),
            pltpu.VMEM((128,), jnp.float32),
            pltpu.VMEM((128,), jnp.float32),
            pltpu.VMEM((_N,), jnp.float32),
            pltpu.VMEM((_N,), jnp.int32),
            pltpu.VMEM((128,), jnp.int32),
            pltpu.VMEM((128,), jnp.float32),
            pltpu.VMEM((128,), jnp.float32),
            pltpu.VMEM((128,), jnp.float32),
            pltpu.VMEM((128,), jnp.float32),
            pltpu.VMEM((_L,), jnp.float32),
        ],
    )
    return f(x, y, z)


def kernel(adv_pcs):
    x = adv_pcs[:, :, 0]
    y = adv_pcs[:, :, 1]
    z = adv_pcs[:, :, 2]
    parts = _loss_sc(x, y, z)
    return parts.reshape(_B, 2, _L).sum(axis=(1, 2))
